# baseline (speedup 1.0000x reference)
import functools
import math

import jax
import jax.numpy as jnp
from jax import lax
from jax.experimental import pallas as pl
from jax.experimental.pallas import tpu as pltpu
from jax.experimental.pallas import tpu_sc as plsc

_VOCAB = 1000000
_D = 64
_B = 4096
_H = 200
_NW = 32
_BB = _B // _NW
_SCALE = math.sqrt(_D)


def kernel(x, lut):
    xT = x.T
    lut_w = lut.reshape(_VOCAB // 2, 2 * _D)
    mesh = plsc.VectorSubcoreMesh(core_axis_name="c", subcore_axis_name="s")

    @functools.partial(
        pl.kernel,
        mesh=mesh,
        out_type=jax.ShapeDtypeStruct((_H, _D, _B), jnp.float32),
        scratch_types=[
            pltpu.VMEM((_H, _BB), jnp.int32),
            pltpu.VMEM((_H, _BB), jnp.int32),
            pltpu.VMEM((_BB, 2 * _D), jnp.float32),
            pltpu.VMEM((_D, _BB), jnp.float32),
            pltpu.SemaphoreType.DMA,
        ],
        compiler_params=pltpu.CompilerParams(needs_layout_passes=False),
    )
    def k(xT_hbm, lut_hbm, out_hbm, idx_v, gidx_v, wide_v, obuf_v, sem):
        wid = lax.axis_index("s") * 2 + lax.axis_index("c")
        b0 = wid * _BB
        pltpu.sync_copy(xT_hbm.at[:, pl.ds(b0, _BB)], idx_v)

        def to_gidx(i, carry):
            r = i // (_BB // 16)
            c = (i % (_BB // 16)) * 16
            gidx_v[r, pl.ds(c, 16)] = lax.shift_right_logical(
                idx_v[r, pl.ds(c, 16)], 1)
            return carry

        lax.fori_loop(0, _H * (_BB // 16), to_gidx, 0, unroll=8)

        lanes = lax.broadcasted_iota(jnp.int32, (16,), 0)

        def per_h(h, carry):
            pltpu.async_copy(lut_hbm.at[gidx_v.at[h]], wide_v, sem).wait()
            for grp in range(_BB // 16):
                bb = grp * 16
                v16 = idx_v[h, pl.ds(bb, 16)]
                col0 = lax.shift_left(lax.bitwise_and(v16, 1), 6)
                rows = lanes + bb
                for d in range(_D):
                    val = plsc.load_gather(wide_v, [rows, col0 + d])
                    obuf_v[d, pl.ds(bb, 16)] = val * _SCALE
            pltpu.sync_copy(obuf_v, out_hbm.at[h, :, pl.ds(b0, _BB)])
            return carry

        lax.fori_loop(0, _H, per_h, 0)

    out3 = k(xT, lut_w)
    return jnp.transpose(out3, (2, 0, 1))


# speedup vs baseline: 1.1969x; 1.1969x over previous
import functools
import math

import jax
import jax.numpy as jnp
from jax import lax
from jax.experimental import pallas as pl
from jax.experimental.pallas import tpu as pltpu
from jax.experimental.pallas import tpu_sc as plsc

_VOCAB = 1000000
_D = 64
_B = 4096
_H = 200
_NW = 32
_BB = _B // _NW
_NG = _BB // 16
_SCALE = math.sqrt(_D)


def kernel(x, lut):
    xT = x.T
    lut_w = lut.reshape(_VOCAB // 2, 2 * _D)
    mesh = plsc.VectorSubcoreMesh(core_axis_name="c", subcore_axis_name="s")

    @functools.partial(
        pl.kernel,
        mesh=mesh,
        out_type=jax.ShapeDtypeStruct((_H, _D, _B), jnp.float32),
        scratch_types=[
            pltpu.VMEM((_H, _BB), jnp.int32),
            pltpu.VMEM((_H, _BB), jnp.int32),
            [pltpu.VMEM((_BB, 2 * _D), jnp.float32) for _ in range(2)],
            [pltpu.VMEM((_D, _BB), jnp.float32) for _ in range(2)],
            [pltpu.SemaphoreType.DMA for _ in range(2)],
            [pltpu.SemaphoreType.DMA for _ in range(2)],
        ],
        compiler_params=pltpu.CompilerParams(needs_layout_passes=False),
    )
    def k(xT_hbm, lut_hbm, out_hbm, idx_v, gidx_v, wides, obufs, gsems,
          ssems):
        wid = lax.axis_index("s") * 2 + lax.axis_index("c")
        b0 = wid * _BB
        pltpu.sync_copy(xT_hbm.at[:, pl.ds(b0, _BB)], idx_v)

        def to_gidx(i, carry):
            r = i // _NG
            c = (i % _NG) * 16
            gidx_v[r, pl.ds(c, 16)] = lax.shift_right_logical(
                idx_v[r, pl.ds(c, 16)], 1)
            return carry

        lax.fori_loop(0, _H * _NG, to_gidx, 0, unroll=8)

        lanes = lax.broadcasted_iota(jnp.int32, (16,), 0)
        rowvec = [lanes + g * 16 for g in range(_NG)]

        def gather(h, p):
            pltpu.async_copy(lut_hbm.at[gidx_v.at[h]], wides[p], gsems[p])

        gather(0, 0)

        def per_h2(h2, carry):
            for p in range(2):
                h = 2 * h2 + p
                wide_v = wides[p]
                obuf_v = obufs[p]

                @pl.when(h + 1 < _H)
                def _():
                    gather(h + 1, 1 - p)

                pltpu.make_async_copy(lut_hbm.at[gidx_v.at[h]], wide_v,
                                      gsems[p]).wait()

                @pl.when(h >= 2)
                def _():
                    pltpu.make_async_copy(
                        obuf_v, out_hbm.at[h - 2, :, pl.ds(b0, _BB)],
                        ssems[p]).wait()

                col0 = [
                    lax.shift_left(
                        lax.bitwise_and(idx_v[h, pl.ds(g * 16, 16)], 1), 6)
                    for g in range(_NG)
                ]

                def per_d(d, c2):
                    for g in range(_NG):
                        val = plsc.load_gather(
                            wide_v, [rowvec[g], col0[g] + d])
                        obuf_v[d, pl.ds(g * 16, 16)] = val * _SCALE
                    return c2

                lax.fori_loop(0, _D, per_d, 0, unroll=2)
                pltpu.async_copy(obuf_v, out_hbm.at[h, :, pl.ds(b0, _BB)],
                                 ssems[p])
            return carry

        lax.fori_loop(0, _H // 2, per_h2, 0)

        for h in (_H - 2, _H - 1):
            p = h % 2
            pltpu.make_async_copy(obufs[p], out_hbm.at[h, :, pl.ds(b0, _BB)],
                                  ssems[p]).wait()

    out3 = k(xT, lut_w)
    return jnp.transpose(out3, (2, 0, 1))


# speedup vs baseline: 1.4445x; 1.2068x over previous
import functools
import math

import jax
import jax.numpy as jnp
from jax import lax
from jax.experimental import pallas as pl
from jax.experimental.pallas import tpu as pltpu
from jax.experimental.pallas import tpu_sc as plsc

_VOCAB = 1000000
_D = 64
_B = 4096
_H = 200
_NW = 32
_BB = _B // _NW
_NG = _BB // 16
_SCALE = math.sqrt(_D)


def kernel(x, lut):
    xT = x.T
    lut_w = lut.reshape(_VOCAB // 2, 2 * _D)
    mesh = plsc.VectorSubcoreMesh(core_axis_name="c", subcore_axis_name="s")

    @functools.partial(
        pl.kernel,
        mesh=mesh,
        out_type=jax.ShapeDtypeStruct((_H, _D, _B), jnp.float32),
        scratch_types=[
            pltpu.VMEM((_H, _BB), jnp.int32),
            pltpu.VMEM((_H, _BB), jnp.int32),
            [pltpu.VMEM((_BB, 2 * _D), jnp.float32) for _ in range(2)],
            [pltpu.VMEM((_D, _BB), jnp.float32) for _ in range(2)],
            [pltpu.SemaphoreType.DMA for _ in range(2)],
            [pltpu.SemaphoreType.DMA for _ in range(2)],
        ],
        compiler_params=pltpu.CompilerParams(needs_layout_passes=False),
    )
    def k(xT_hbm, lut_hbm, out_hbm, idx_v, gidx_v, wides, obufs, gsems,
          ssems):
        wid = lax.axis_index("s") * 2 + lax.axis_index("c")
        b0 = wid * _BB
        pltpu.sync_copy(xT_hbm.at[:, pl.ds(b0, _BB)], idx_v)

        def to_gidx(i, carry):
            r = i // _NG
            c = (i % _NG) * 16
            gidx_v[r, pl.ds(c, 16)] = lax.shift_right_logical(
                idx_v[r, pl.ds(c, 16)], 1)
            return carry

        lax.fori_loop(0, _H * _NG, to_gidx, 0, unroll=8)

        lanes = lax.broadcasted_iota(jnp.int32, (16,), 0)
        zeros16 = lanes * 0
        rowbase = [(lanes + g * 16) * (2 * _D) for g in range(_NG)]

        def gather(h, p):
            pltpu.async_copy(lut_hbm.at[gidx_v.at[h]], wides[p], gsems[p])

        gather(0, 0)

        def per_h2(h2, carry):
            for p in range(2):
                h = 2 * h2 + p
                wide_v = wides[p]
                obuf_v = obufs[p]

                @pl.when(h + 1 < _H)
                def _():
                    gather(h + 1, 1 - p)

                pltpu.make_async_copy(lut_hbm.at[gidx_v.at[h]], wide_v,
                                      gsems[p]).wait()

                @pl.when(h >= 2)
                def _():
                    pltpu.make_async_copy(
                        obuf_v, out_hbm.at[h - 2, :, pl.ds(b0, _BB)],
                        ssems[p]).wait()

                base0 = tuple(
                    rowbase[g] + lax.shift_left(
                        lax.bitwise_and(idx_v[h, pl.ds(g * 16, 16)], 1), 6)
                    for g in range(_NG)
                )

                def per_d(d, bases):
                    vals = [plsc.load_gather(wide_v, [zeros16, bases[g]])
                            for g in range(_NG)]
                    for g in range(_NG):
                        obuf_v[d, pl.ds(g * 16, 16)] = vals[g] * _SCALE
                    return tuple(b + 1 for b in bases)

                lax.fori_loop(0, _D, per_d, base0, unroll=2)
                pltpu.async_copy(obuf_v, out_hbm.at[h, :, pl.ds(b0, _BB)],
                                 ssems[p])
            return carry

        lax.fori_loop(0, _H // 2, per_h2, 0)

        for h in (_H - 2, _H - 1):
            p = h % 2
            pltpu.make_async_copy(obufs[p], out_hbm.at[h, :, pl.ds(b0, _BB)],
                                  ssems[p]).wait()

    out3 = k(xT, lut_w)
    return jnp.transpose(out3, (2, 0, 1))


# speedup vs baseline: 2.7818x; 1.9258x over previous
import functools
import math

import jax
import jax.numpy as jnp
from jax import lax
from jax.experimental import pallas as pl
from jax.experimental.pallas import tpu as pltpu
from jax.experimental.pallas import tpu_sc as plsc

_VOCAB = 1000000
_D = 64
_B = 4096
_H = 200
_NW = 32
_BB = _B // _NW
_NG = _BB // 16
_SCALE = math.sqrt(_D)


def kernel(x, lut):
    xT = x.T
    lut_w = lut.reshape(_VOCAB // 2, 2 * _D)
    mesh = plsc.VectorSubcoreMesh(core_axis_name="c", subcore_axis_name="s")

    @functools.partial(
        pl.kernel,
        mesh=mesh,
        out_type=jax.ShapeDtypeStruct((_H, _D, _B), jnp.float32),
        scratch_types=[
            pltpu.VMEM((_H, _BB), jnp.int32),
            pltpu.VMEM((_H, _BB), jnp.int32),
            [pltpu.VMEM((_BB, 2 * _D), jnp.float32) for _ in range(2)],
            [pltpu.VMEM((_D, _BB), jnp.float32) for _ in range(2)],
            [pltpu.SemaphoreType.DMA for _ in range(2)],
            [pltpu.SemaphoreType.DMA for _ in range(2)],
        ],
        compiler_params=pltpu.CompilerParams(needs_layout_passes=False),
    )
    def k(xT_hbm, lut_hbm, out_hbm, idx_v, gidx_v, wides, obufs, gsems,
          ssems):
        wid = lax.axis_index("s") * 2 + lax.axis_index("c")
        b0 = wid * _BB
        pltpu.sync_copy(xT_hbm.at[:, pl.ds(b0, _BB)], idx_v)

        def to_gidx(i, carry):
            r = i // _NG
            c = (i % _NG) * 16
            gidx_v[r, pl.ds(c, 16)] = lax.shift_right_logical(
                idx_v[r, pl.ds(c, 16)], 1)
            return carry

        lax.fori_loop(0, _H * _NG, to_gidx, 0, unroll=8)

        lanes = lax.broadcasted_iota(jnp.int32, (16,), 0)
        zeros16 = lanes * 0
        rowbase = [(lanes + g * 16) * (2 * _D) for g in range(_NG)]

        def gather(h, p):
            pltpu.async_copy(lut_hbm.at[gidx_v.at[h]], wides[p], gsems[p])

        gather(0, 0)

        def per_h2(h2, carry):
            for p in range(2):
                h = 2 * h2 + p
                wide_v = wides[p]
                obuf_v = obufs[p]

                @pl.when(h + 1 < _H)
                def _():
                    gather(h + 1, 1 - p)

                pltpu.make_async_copy(lut_hbm.at[gidx_v.at[h]], wide_v,
                                      gsems[p]).wait()

                @pl.when(h >= 2)
                def _():
                    pltpu.make_async_copy(
                        obuf_v, out_hbm.at[h - 2, :, pl.ds(b0, _BB)],
                        ssems[p]).wait()

                base0 = tuple(
                    rowbase[g] + lax.shift_left(
                        lax.bitwise_and(idx_v[h, pl.ds(g * 16, 16)], 1), 6)
                    for g in range(_NG)
                )

                @functools.partial(plsc.parallel_loop, 0, _D, unroll=4)
                def _(d):
                    vals = [plsc.load_gather(wide_v, [zeros16, base0[g] + d])
                            for g in range(_NG)]
                    for g in range(_NG):
                        obuf_v[d, pl.ds(g * 16, 16)] = vals[g] * _SCALE
                pltpu.async_copy(obuf_v, out_hbm.at[h, :, pl.ds(b0, _BB)],
                                 ssems[p])
            return carry

        lax.fori_loop(0, _H // 2, per_h2, 0)

        for h in (_H - 2, _H - 1):
            p = h % 2
            pltpu.make_async_copy(obufs[p], out_hbm.at[h, :, pl.ds(b0, _BB)],
                                  ssems[p]).wait()

    out3 = k(xT, lut_w)
    return jnp.transpose(out3, (2, 0, 1))
